# baseline (speedup 1.0000x reference)
import functools

import jax
import jax.numpy as jnp
from jax import lax
from jax.experimental import pallas as pl
from jax.experimental.pallas import tpu as pltpu
from jax.experimental.pallas import tpu_sc as plsc

V, D = 1_000_000, 32
B, L = 16384, 50
NTOT = B * L
NC, NS = 2, 16
NW = NC * NS
PER_W = NTOT // NW
CHUNK = 512
G = CHUNK // 128
NCHUNK = PER_W // CHUNK
NTJ = NTOT // 128
OROWS = NTOT * D // 128


def _sc_gather(idx_flat, table):
    mesh = plsc.VectorSubcoreMesh(core_axis_name="c", subcore_axis_name="s")

    @functools.partial(
        pl.kernel,
        out_type=jax.ShapeDtypeStruct((OROWS, 128), jnp.float32),
        mesh=mesh,
        scratch_types=[
            pltpu.VMEM((2, CHUNK), jnp.int32),
            pltpu.VMEM((2, CHUNK, D), jnp.float32),
            pltpu.VMEM((2, D, G * 128 + 1), jnp.float32),
            pltpu.SemaphoreType.DMA,
            pltpu.SemaphoreType.DMA,
            pltpu.SemaphoreType.DMA,
            pltpu.SemaphoreType.DMA,
        ],
        compiler_params=pltpu.CompilerParams(use_tc_tiling_on_sc=False,
                                             needs_layout_passes=False,
                                             skip_device_barrier=True),
    )
    def k(idx_hbm, table_hbm, out_hbm, idx_v, rows_v, stage_v,
          gsem0, gsem1, osem0, osem1):
        gsem = (gsem0, gsem1)
        osem = (osem0, osem1)
        wid = lax.axis_index("s") * NC + lax.axis_index("c")
        wbase = wid * PER_W
        lane = lax.iota(jnp.int32, 16)

        def fire_gathers(buf, chunk):
            base = wbase + chunk * CHUNK
            pltpu.sync_copy(idx_hbm.at[pl.ds(base, CHUNK)], idx_v.at[buf])
            for j in range(G):
                pltpu.async_copy(
                    table_hbm.at[idx_v.at[buf].at[pl.ds(j * 128, 128)]],
                    rows_v.at[buf].at[pl.ds(j * 128, 128)], gsem[buf])

        def wait_gathers(buf):
            for j in range(G):
                pltpu.make_async_copy(
                    table_hbm.at[idx_v.at[buf].at[pl.ds(j * 128, 128)]],
                    rows_v.at[buf].at[pl.ds(j * 128, 128)], gsem[buf]).wait()

        dlo = lane
        dhi = lane + 16

        def transpose(buf):
            @plsc.parallel_loop(0, CHUNK, unroll=4)
            def tent(e):
                ev = jnp.full((16,), 0, jnp.int32) + e
                v0 = rows_v.at[buf][e, pl.ds(0, 16)]
                plsc.store_scatter(stage_v.at[buf], [dlo, ev], v0)
                v1 = rows_v.at[buf][e, pl.ds(16, 16)]
                plsc.store_scatter(stage_v.at[buf], [dhi, ev], v1)

        def fire_writeback(buf, chunk):
            tj0 = (wbase + chunk * CHUNK) // 128
            for ci in range(4):
                for t in range(G):
                    pltpu.async_copy(
                        stage_v.at[buf].at[pl.ds(ci * 8, 8),
                                           pl.ds(t * 128, 128)],
                        out_hbm.at[pl.ds(8 * (ci * NTJ + tj0 + t), 8)],
                        osem[buf])

        def wait_writeback(buf, chunk):
            tj0 = (wbase + chunk * CHUNK) // 128
            for ci in range(4):
                for t in range(G):
                    pltpu.make_async_copy(
                        stage_v.at[buf].at[pl.ds(ci * 8, 8),
                                           pl.ds(t * 128, 128)],
                        out_hbm.at[pl.ds(8 * (ci * NTJ + tj0 + t), 8)],
                        osem[buf]).wait()

        fire_gathers(0, 0)

        def body(i2, _):
            for b in range(2):
                i = 2 * i2 + b
                nbuf = 1 - b

                @pl.when(i >= 1)
                def _():
                    wait_writeback(nbuf, i - 1)

                @pl.when(i + 1 < NCHUNK)
                def _():
                    fire_gathers(nbuf, i + 1)

                wait_gathers(b)
                transpose(b)
                fire_writeback(b, i)
            return ()

        lax.fori_loop(0, NCHUNK // 2, body, (), unroll=False)
        wait_writeback((NCHUNK - 1) % 2, NCHUNK - 1)

    return k(idx_flat, table)


def kernel(idx, table):
    t = _sc_gather(idx.reshape(-1), table)
    return t.reshape(4, NTJ, 8, 128).transpose(1, 3, 0, 2).reshape(NTOT, D)


# speedup vs baseline: 1.0111x; 1.0111x over previous
import functools

import jax
import jax.numpy as jnp
from jax import lax
from jax.experimental import pallas as pl
from jax.experimental.pallas import tpu as pltpu
from jax.experimental.pallas import tpu_sc as plsc

V, D = 1_000_000, 32
B, L = 16384, 50
NTOT = B * L
NC, NS = 2, 16
NW = NC * NS
PER_W = NTOT // NW
CHUNK = 640
G = CHUNK // 128
NCHUNK = PER_W // CHUNK
NTJ = NTOT // 128
OROWS = NTOT * D // 128


def _sc_gather(idx_flat, table):
    mesh = plsc.VectorSubcoreMesh(core_axis_name="c", subcore_axis_name="s")

    @functools.partial(
        pl.kernel,
        out_type=jax.ShapeDtypeStruct((OROWS, 128), jnp.float32),
        mesh=mesh,
        scratch_types=[
            pltpu.VMEM((2, CHUNK), jnp.int32),
            pltpu.VMEM((2, CHUNK, D), jnp.float32),
            pltpu.VMEM((2, D, G * 128 + 1), jnp.float32),
            pltpu.SemaphoreType.DMA,
            pltpu.SemaphoreType.DMA,
            pltpu.SemaphoreType.DMA,
            pltpu.SemaphoreType.DMA,
        ],
        compiler_params=pltpu.CompilerParams(use_tc_tiling_on_sc=False,
                                             needs_layout_passes=False),
    )
    def k(idx_hbm, table_hbm, out_hbm, idx_v, rows_v, stage_v,
          gsem0, gsem1, osem0, osem1):
        gsem = (gsem0, gsem1)
        osem = (osem0, osem1)
        wid = lax.axis_index("s") * NC + lax.axis_index("c")
        wbase = wid * PER_W
        lane = lax.iota(jnp.int32, 16)

        def fire_gathers(buf, chunk):
            base = wbase + chunk * CHUNK
            pltpu.sync_copy(idx_hbm.at[pl.ds(base, CHUNK)], idx_v.at[buf])
            for j in range(G):
                pltpu.async_copy(
                    table_hbm.at[idx_v.at[buf].at[pl.ds(j * 128, 128)]],
                    rows_v.at[buf].at[pl.ds(j * 128, 128)], gsem[buf])

        def wait_gathers(buf):
            for j in range(G):
                pltpu.make_async_copy(
                    table_hbm.at[idx_v.at[buf].at[pl.ds(j * 128, 128)]],
                    rows_v.at[buf].at[pl.ds(j * 128, 128)], gsem[buf]).wait()

        dlo = lane
        dhi = lane + 16

        def transpose(buf):
            @plsc.parallel_loop(0, CHUNK, unroll=4)
            def tent(e):
                ev = jnp.full((16,), 0, jnp.int32) + e
                v0 = rows_v.at[buf][e, pl.ds(0, 16)]
                plsc.store_scatter(stage_v.at[buf], [dlo, ev], v0)
                v1 = rows_v.at[buf][e, pl.ds(16, 16)]
                plsc.store_scatter(stage_v.at[buf], [dhi, ev], v1)

        def fire_writeback(buf, chunk):
            tj0 = (wbase + chunk * CHUNK) // 128
            for ci in range(4):
                for t in range(G):
                    pltpu.async_copy(
                        stage_v.at[buf].at[pl.ds(ci * 8, 8),
                                           pl.ds(t * 128, 128)],
                        out_hbm.at[pl.ds(8 * (ci * NTJ + tj0 + t), 8)],
                        osem[buf])

        def wait_writeback(buf, chunk):
            tj0 = (wbase + chunk * CHUNK) // 128
            for ci in range(4):
                for t in range(G):
                    pltpu.make_async_copy(
                        stage_v.at[buf].at[pl.ds(ci * 8, 8),
                                           pl.ds(t * 128, 128)],
                        out_hbm.at[pl.ds(8 * (ci * NTJ + tj0 + t), 8)],
                        osem[buf]).wait()

        fire_gathers(0, 0)

        def body(i2, _):
            for b in range(2):
                i = 2 * i2 + b
                nbuf = 1 - b

                @pl.when(i >= 1)
                def _():
                    wait_writeback(nbuf, i - 1)

                @pl.when(i + 1 < NCHUNK)
                def _():
                    fire_gathers(nbuf, i + 1)

                wait_gathers(b)
                transpose(b)
                fire_writeback(b, i)
            return ()

        lax.fori_loop(0, NCHUNK // 2, body, (), unroll=False)
        wait_writeback((NCHUNK - 1) % 2, NCHUNK - 1)

    return k(idx_flat, table)


def kernel(idx, table):
    t = _sc_gather(idx.reshape(-1), table)
    return t.reshape(4, NTJ, 8, 128).transpose(1, 3, 0, 2).reshape(NTOT, D)
